# baseline (speedup 1.0000x reference)
import functools

import jax
import jax.numpy as jnp
from jax import lax
from jax.experimental import pallas as pl
from jax.experimental.pallas import tpu as pltpu
from jax.experimental.pallas import tpu_sc as plsc

N_EXPERTS = 64
N_GROUP = 8
GROUP_SIZE = N_EXPERTS // N_GROUP
TOPK_GROUP = 4
TOP_K = 8
SCALE = 2.5
NEG_INF = float("-inf")

T_TOKENS = 16384
N_WORKERS = 32
TOK_PER_W = T_TOKENS // N_WORKERS
LANES = 16
NBLK = TOK_PER_W // LANES


def _scores_kernel(x_ref, w_ref, bias_ref, sc_ref, sfc_ref):
    logits = jax.lax.dot_general(
        w_ref[...], x_ref[...],
        dimension_numbers=(((1,), (1,)), ((), ())),
        preferred_element_type=jnp.float32,
        precision=jax.lax.Precision.DEFAULT,
    )
    scores = jax.nn.sigmoid(logits)
    sc_ref[...] = scores
    sfc_ref[...] = scores + bias_ref[...]


def _tree_max(vals):
    vals = list(vals)
    while len(vals) > 1:
        nxt = [jnp.maximum(vals[i], vals[i + 1]) for i in range(0, len(vals) - 1, 2)]
        if len(vals) % 2:
            nxt.append(vals[-1])
        vals = nxt
    return vals[0]


def _first_argmax_list(vals, m):
    am = jnp.full((LANES,), len(vals), dtype=jnp.int32)
    for j in range(len(vals) - 1, -1, -1):
        am = jnp.where(vals[j] == m, jnp.int32(j), am)
    return am


def _sc_route_kernel(sfc_hbm, sc_hbm, idx_hbm, w_hbm, sfc_v, sc_v, idx_v, w_v):
    wid = lax.axis_index("s") * 2 + lax.axis_index("c")
    base = wid * TOK_PER_W
    pltpu.sync_copy(sfc_hbm.at[:, pl.ds(base, TOK_PER_W)], sfc_v)
    pltpu.sync_copy(sc_hbm.at[:, pl.ds(base, TOK_PER_W)], sc_v)

    def body(b, carry):
        off = b * LANES
        sfc = [sfc_v[e, pl.ds(off, LANES)] for e in range(N_EXPERTS)]

        keep = []
        gsum = []
        for g in range(N_GROUP):
            grp = sfc[g * GROUP_SIZE:(g + 1) * GROUP_SIZE]
            m1 = _tree_max(grp)
            am = _first_argmax_list(grp, m1)
            m2 = _tree_max([jnp.where(am == j, NEG_INF, grp[j]) for j in range(GROUP_SIZE)])
            gsum.append(m1 + m2)

        t = list(gsum)
        keep = [jnp.zeros((LANES,), dtype=jnp.bool_) for _ in range(N_GROUP)]
        for _ in range(TOPK_GROUP):
            m = _tree_max(t)
            am = _first_argmax_list(t, m)
            for g in range(N_GROUP):
                sel = am == g
                keep[g] = jnp.logical_or(keep[g], sel)
                t[g] = jnp.where(sel, NEG_INF, t[g])

        tmp = [jnp.where(keep[e // GROUP_SIZE], sfc[e], NEG_INF) for e in range(N_EXPERTS)]

        scv = [sc_v[e, pl.ds(off, LANES)] for e in range(N_EXPERTS)]
        ws = []
        for k in range(TOP_K):
            m = _tree_max(tmp)
            am = _first_argmax_list(tmp, m)
            sel = [am == e for e in range(N_EXPERTS)]
            wk = _tree_max([jnp.where(sel[e], scv[e], NEG_INF) for e in range(N_EXPERTS)])
            for e in range(N_EXPERTS):
                tmp[e] = jnp.where(sel[e], NEG_INF, tmp[e])
            idx_v[k, pl.ds(off, LANES)] = am
            ws.append(wk)

        denom = ws[0]
        for k in range(1, TOP_K):
            denom = denom + ws[k]
        denom = denom + 1e-20
        for k in range(TOP_K):
            w_v[k, pl.ds(off, LANES)] = ws[k] / denom * SCALE
        return carry

    lax.fori_loop(0, NBLK, body, jnp.int32(0))

    pltpu.sync_copy(idx_v, idx_hbm.at[:, pl.ds(base, TOK_PER_W)])
    pltpu.sync_copy(w_v, w_hbm.at[:, pl.ds(base, TOK_PER_W)])


@jax.jit
def _run(x, weight, bias):
    t = x.shape[0]
    tb = 1024
    grid = (t // tb,)
    scores, sfc = pl.pallas_call(
        _scores_kernel,
        grid=grid,
        in_specs=[
            pl.BlockSpec((tb, x.shape[1]), lambda i: (i, 0)),
            pl.BlockSpec((N_EXPERTS, x.shape[1]), lambda i: (0, 0)),
            pl.BlockSpec((N_EXPERTS, 1), lambda i: (0, 0)),
        ],
        out_specs=[
            pl.BlockSpec((N_EXPERTS, tb), lambda i: (0, i)),
            pl.BlockSpec((N_EXPERTS, tb), lambda i: (0, i)),
        ],
        out_shape=[
            jax.ShapeDtypeStruct((N_EXPERTS, t), jnp.float32),
            jax.ShapeDtypeStruct((N_EXPERTS, t), jnp.float32),
        ],
    )(x, weight, bias)

    mesh = plsc.VectorSubcoreMesh(core_axis_name="c", subcore_axis_name="s")
    idx_t, w_t = pl.kernel(
        _sc_route_kernel,
        mesh=mesh,
        out_type=[
            jax.ShapeDtypeStruct((TOP_K, t), jnp.int32),
            jax.ShapeDtypeStruct((TOP_K, t), jnp.float32),
        ],
        scratch_types=[
            pltpu.VMEM((N_EXPERTS, TOK_PER_W), jnp.float32),
            pltpu.VMEM((N_EXPERTS, TOK_PER_W), jnp.float32),
            pltpu.VMEM((TOP_K, TOK_PER_W), jnp.int32),
            pltpu.VMEM((TOP_K, TOK_PER_W), jnp.float32),
        ],
    )(sfc, scores)
    return idx_t, w_t


def kernel(hidden_states, weight, e_score_correction_bias):
    bsz, seq_len, h = hidden_states.shape
    x = hidden_states.reshape(-1, h).astype(jnp.float32)
    bias = e_score_correction_bias.reshape(N_EXPERTS, 1).astype(jnp.float32)
    idx_t, w_t = _run(x, weight.astype(jnp.float32), bias)
    return idx_t.T, w_t.T


# speedup vs baseline: 2.6051x; 2.6051x over previous
import jax
import jax.numpy as jnp
from jax import lax
from jax.experimental import pallas as pl

N_EXPERTS = 64
N_GROUP = 8
GROUP_SIZE = N_EXPERTS // N_GROUP
TOPK_GROUP = 4
TOP_K = 8
SCALE = 2.5
NEG_INF = float("-inf")


def _first_argmax0(x, row_iota, height):
    m = jnp.max(x, axis=0, keepdims=True)
    am = jnp.min(jnp.where(x == m, row_iota, height), axis=0, keepdims=True)
    return m, am


def _router_kernel(x_ref, w_ref, bias_ref, idx_ref, w_out_ref):
    tb = x_ref.shape[0]
    logits = jax.lax.dot_general(
        w_ref[...], x_ref[...],
        dimension_numbers=(((1,), (1,)), ((), ())),
        preferred_element_type=jnp.float32,
        precision=jax.lax.Precision.DEFAULT,
    )
    scores = jax.nn.sigmoid(logits)
    sfc = scores + bias_ref[...]

    io_gs = lax.broadcasted_iota(jnp.int32, (GROUP_SIZE, tb), 0)
    io8 = lax.broadcasted_iota(jnp.int32, (N_GROUP, tb), 0)
    io64 = lax.broadcasted_iota(jnp.int32, (N_EXPERTS, tb), 0)

    rows = []
    for g in range(N_GROUP):
        blk = sfc[g * GROUP_SIZE:(g + 1) * GROUP_SIZE, :]
        m1, am = _first_argmax0(blk, io_gs, GROUP_SIZE)
        m2 = jnp.max(jnp.where(io_gs == am, NEG_INF, blk), axis=0, keepdims=True)
        rows.append(m1 + m2)
    group_scores = jnp.concatenate(rows, axis=0)

    gmask = jnp.zeros((N_GROUP, tb), dtype=jnp.bool_)
    gs = group_scores
    for _ in range(TOPK_GROUP):
        _, am = _first_argmax0(gs, io8, N_GROUP)
        sel = io8 == am
        gmask = jnp.logical_or(gmask, sel)
        gs = jnp.where(sel, NEG_INF, gs)

    blocks = [
        jnp.where(gmask[g:g + 1, :], sfc[g * GROUP_SIZE:(g + 1) * GROUP_SIZE, :], NEG_INF)
        for g in range(N_GROUP)
    ]
    tmp = jnp.concatenate(blocks, axis=0)
    idx_rows = []
    w_rows = []
    for _ in range(TOP_K):
        _, am = _first_argmax0(tmp, io64, N_EXPERTS)
        sel = io64 == am
        idx_rows.append(am)
        w_rows.append(jnp.max(jnp.where(sel, scores, NEG_INF), axis=0, keepdims=True))
        tmp = jnp.where(sel, NEG_INF, tmp)
    topk_idx = jnp.concatenate(idx_rows, axis=0)
    topk_w = jnp.concatenate(w_rows, axis=0)

    denom = jnp.sum(topk_w, axis=0, keepdims=True) + 1e-20
    idx_ref[...] = topk_idx
    w_out_ref[...] = topk_w / denom * SCALE


@jax.jit
def _run(x, weight, bias):
    t = x.shape[0]
    tb = 1024
    grid = (t // tb,)
    return pl.pallas_call(
        _router_kernel,
        grid=grid,
        in_specs=[
            pl.BlockSpec((tb, x.shape[1]), lambda i: (i, 0)),
            pl.BlockSpec((N_EXPERTS, x.shape[1]), lambda i: (0, 0)),
            pl.BlockSpec((N_EXPERTS, 1), lambda i: (0, 0)),
        ],
        out_specs=[
            pl.BlockSpec((TOP_K, tb), lambda i: (0, i)),
            pl.BlockSpec((TOP_K, tb), lambda i: (0, i)),
        ],
        out_shape=[
            jax.ShapeDtypeStruct((TOP_K, t), jnp.int32),
            jax.ShapeDtypeStruct((TOP_K, t), jnp.float32),
        ],
    )(x, weight, bias)


def kernel(hidden_states, weight, e_score_correction_bias):
    bsz, seq_len, h = hidden_states.shape
    x = hidden_states.reshape(-1, h).astype(jnp.float32)
    bias = e_score_correction_bias.reshape(N_EXPERTS, 1).astype(jnp.float32)
    idx_t, w_t = _run(x, weight.astype(jnp.float32), bias)
    return idx_t.T, w_t.T
